# baseline (speedup 1.0000x reference)
import jax
import jax.numpy as jnp
from jax.experimental import pallas as pl
from jax.experimental.pallas import tpu as pltpu

_B, _N, _D_IN = 8, 2048, 1408
_PROJ = 256
_CB_SIZES = (64, 128, 256)
_CC = 0.25
_M = 512
_G = (_B * _N) // _M
_LOSS_SCALE = _CC / (3.0 * (_B * _N) * _PROJ)

_PREC = jax.lax.Precision.DEFAULT


def _vq_kernel(z_ref, w_in_ref, b_in_ref, w_out_ref, b_out_ref,
               e0_ref, e1_ref, e2_ref,
               out_ref, i0_ref, i1_ref, i2_ref, loss_ref):
    i = pl.program_id(0)
    x = z_ref[...]
    zp = jax.lax.dot_general(
        x, w_in_ref[...], (((1,), (1,)), ((), ())),
        precision=_PREC, preferred_element_type=jnp.float32)
    zp = zp + b_in_ref[...]

    residual = zp
    zq_sum = jnp.zeros_like(zp)
    part = jnp.float32(0.0)
    for e_ref, idx_ref in ((e0_ref, i0_ref), (e1_ref, i1_ref), (e2_ref, i2_ref)):
        e = e_ref[...]
        k = e.shape[0]
        cross = jax.lax.dot_general(
            residual, e, (((1,), (1,)), ((), ())),
            precision=_PREC, preferred_element_type=jnp.float32)
        en = jnp.sum(e ** 2, axis=1)
        sq = jnp.sum(residual ** 2, axis=1, keepdims=True)
        dist = (sq - 2.0 * cross) + en[None, :]
        dmin = jnp.min(dist, axis=1, keepdims=True)
        iota_f = jax.lax.broadcasted_iota(jnp.int32, (_M, k), 1).astype(jnp.float32)
        idx_f = jnp.min(jnp.where(dist == dmin, iota_f, jnp.float32(k)),
                        axis=1, keepdims=True)
        idx = idx_f[:, 0].astype(jnp.int32)
        onehot = (iota_f == idx_f).astype(jnp.bfloat16)
        mask = jnp.uint32(0xFFFF0000)
        ebits = jax.lax.bitcast_convert_type(e, jnp.uint32)
        hi = jax.lax.bitcast_convert_type(ebits & mask, jnp.float32)
        rem = e - hi
        rbits = jax.lax.bitcast_convert_type(rem, jnp.uint32)
        mid = jax.lax.bitcast_convert_type(rbits & mask, jnp.float32)
        lo = rem - mid
        e3 = jnp.concatenate(
            [hi.astype(jnp.bfloat16), mid.astype(jnp.bfloat16),
             lo.astype(jnp.bfloat16)], axis=0)
        oh3 = jnp.concatenate([onehot, onehot, onehot], axis=1)
        zq = jax.lax.dot_general(
            oh3, e3, (((1,), (0,)), ((), ())),
            precision=_PREC, preferred_element_type=jnp.float32)

        diff = zq - residual
        part += jnp.sum(diff * diff)
        idx_ref[0, 0, :] = idx
        zq_sum = zq_sum + zq
        residual = residual - zq

    out = jax.lax.dot_general(
        zq_sum.astype(jnp.bfloat16), w_out_ref[...].astype(jnp.bfloat16),
        (((1,), (1,)), ((), ())),
        precision=_PREC, preferred_element_type=jnp.float32)
    out_ref[...] = out + b_out_ref[...]

    prev = jnp.where(i == 0, jnp.zeros((1, 1), jnp.float32), loss_ref[...])
    loss_ref[...] = prev + part * _LOSS_SCALE


@jax.jit
def _run(z2d, w_in, b_in2d, w_out, b_out2d, e0, e1, e2):
    out, i0, i1, i2, loss = pl.pallas_call(
        _vq_kernel,
        grid=(_G,),
        in_specs=[
            pl.BlockSpec((_M, _D_IN), lambda i: (i, 0)),
            pl.BlockSpec((_PROJ, _D_IN), lambda i: (0, 0)),
            pl.BlockSpec((1, _PROJ), lambda i: (0, 0)),
            pl.BlockSpec((_D_IN, _PROJ), lambda i: (0, 0)),
            pl.BlockSpec((1, _D_IN), lambda i: (0, 0)),
            pl.BlockSpec((_CB_SIZES[0], _PROJ), lambda i: (0, 0)),
            pl.BlockSpec((_CB_SIZES[1], _PROJ), lambda i: (0, 0)),
            pl.BlockSpec((_CB_SIZES[2], _PROJ), lambda i: (0, 0)),
        ],
        out_specs=[
            pl.BlockSpec((_M, _D_IN), lambda i: (i, 0)),
            pl.BlockSpec((1, 1, _M), lambda i: (i, 0, 0)),
            pl.BlockSpec((1, 1, _M), lambda i: (i, 0, 0)),
            pl.BlockSpec((1, 1, _M), lambda i: (i, 0, 0)),
            pl.BlockSpec((1, 1), lambda i: (0, 0)),
        ],
        compiler_params=pltpu.CompilerParams(
            vmem_limit_bytes=100 * 1024 * 1024),
        out_shape=[
            jax.ShapeDtypeStruct((_B * _N, _D_IN), jnp.float32),
            jax.ShapeDtypeStruct((_G, 1, _M), jnp.int32),
            jax.ShapeDtypeStruct((_G, 1, _M), jnp.int32),
            jax.ShapeDtypeStruct((_G, 1, _M), jnp.int32),
            jax.ShapeDtypeStruct((1, 1), jnp.float32),
        ],
    )(z2d, w_in, b_in2d, w_out, b_out2d, e0, e1, e2)
    return out, i0, i1, i2, loss


def kernel(z, W_in, b_in, W_out, b_out, emb0, emb1, emb2):
    z2d = z.reshape(_B * _N, _D_IN)
    out, i0, i1, i2, loss = _run(
        z2d, W_in, b_in.reshape(1, _PROJ), W_out, b_out.reshape(1, _D_IN),
        emb0, emb1, emb2)
    return (out.reshape(_B, _N, _D_IN),
            i0.reshape(_B, _N), i1.reshape(_B, _N), i2.reshape(_B, _N),
            loss[0, 0])


# speedup vs baseline: 1.2399x; 1.2399x over previous
import jax
import jax.numpy as jnp
from jax.experimental import pallas as pl
from jax.experimental.pallas import tpu as pltpu

_B, _N, _D_IN = 8, 2048, 1408
_PROJ = 256
_CB_SIZES = (64, 128, 256)
_CC = 0.25
_M = 1024
_G = (_B * _N) // _M
_LOSS_SCALE = _CC / (3.0 * (_B * _N) * _PROJ)

_PREC = jax.lax.Precision.DEFAULT


def _vq_kernel(z_ref, w_in_ref, b_in_ref, w_out_ref, b_out_ref,
               e0_ref, e1_ref, e2_ref,
               out_ref, i0_ref, i1_ref, i2_ref, loss_ref):
    i = pl.program_id(0)
    x = z_ref[...]
    zp = jax.lax.dot_general(
        x, w_in_ref[...], (((1,), (1,)), ((), ())),
        precision=_PREC, preferred_element_type=jnp.float32)
    zp = zp + b_in_ref[...]

    residual = zp
    zq_sum = jnp.zeros_like(zp)
    part = jnp.float32(0.0)
    for e_ref, idx_ref in ((e0_ref, i0_ref), (e1_ref, i1_ref), (e2_ref, i2_ref)):
        e = e_ref[...]
        k = e.shape[0]
        cross = jax.lax.dot_general(
            residual, e, (((1,), (1,)), ((), ())),
            precision=_PREC, preferred_element_type=jnp.float32)
        en = jnp.sum(e ** 2, axis=1)
        sq = jnp.sum(residual ** 2, axis=1, keepdims=True)
        dist = (sq - 2.0 * cross) + en[None, :]
        dmin = jnp.min(dist, axis=1, keepdims=True)
        iota_f = jax.lax.broadcasted_iota(jnp.int32, (_M, k), 1).astype(jnp.float32)
        idx_f = jnp.min(jnp.where(dist == dmin, iota_f, jnp.float32(k)),
                        axis=1, keepdims=True)
        idx = idx_f[:, 0].astype(jnp.int32)
        onehot = (iota_f == idx_f).astype(jnp.bfloat16)
        mask = jnp.uint32(0xFFFF0000)
        ebits = jax.lax.bitcast_convert_type(e, jnp.uint32)
        hi = jax.lax.bitcast_convert_type(ebits & mask, jnp.float32)
        rem = e - hi
        rbits = jax.lax.bitcast_convert_type(rem, jnp.uint32)
        mid = jax.lax.bitcast_convert_type(rbits & mask, jnp.float32)
        lo = rem - mid
        e3 = jnp.concatenate(
            [hi.astype(jnp.bfloat16), mid.astype(jnp.bfloat16),
             lo.astype(jnp.bfloat16)], axis=0)
        oh3 = jnp.concatenate([onehot, onehot, onehot], axis=1)
        zq = jax.lax.dot_general(
            oh3, e3, (((1,), (0,)), ((), ())),
            precision=_PREC, preferred_element_type=jnp.float32)

        diff = zq - residual
        part += jnp.sum(diff * diff)
        idx_ref[0, 0, :] = idx
        zq_sum = zq_sum + zq
        residual = residual - zq

    out = jax.lax.dot_general(
        zq_sum.astype(jnp.bfloat16), w_out_ref[...].astype(jnp.bfloat16),
        (((1,), (1,)), ((), ())),
        precision=_PREC, preferred_element_type=jnp.float32)
    out_ref[...] = out + b_out_ref[...]

    prev = jnp.where(i == 0, jnp.zeros((1, 1), jnp.float32), loss_ref[...])
    loss_ref[...] = prev + part * _LOSS_SCALE


@jax.jit
def _run(z2d, w_in, b_in2d, w_out, b_out2d, e0, e1, e2):
    out, i0, i1, i2, loss = pl.pallas_call(
        _vq_kernel,
        grid=(_G,),
        in_specs=[
            pl.BlockSpec((_M, _D_IN), lambda i: (i, 0)),
            pl.BlockSpec((_PROJ, _D_IN), lambda i: (0, 0)),
            pl.BlockSpec((1, _PROJ), lambda i: (0, 0)),
            pl.BlockSpec((_D_IN, _PROJ), lambda i: (0, 0)),
            pl.BlockSpec((1, _D_IN), lambda i: (0, 0)),
            pl.BlockSpec((_CB_SIZES[0], _PROJ), lambda i: (0, 0)),
            pl.BlockSpec((_CB_SIZES[1], _PROJ), lambda i: (0, 0)),
            pl.BlockSpec((_CB_SIZES[2], _PROJ), lambda i: (0, 0)),
        ],
        out_specs=[
            pl.BlockSpec((_M, _D_IN), lambda i: (i, 0)),
            pl.BlockSpec((1, 1, _M), lambda i: (i, 0, 0)),
            pl.BlockSpec((1, 1, _M), lambda i: (i, 0, 0)),
            pl.BlockSpec((1, 1, _M), lambda i: (i, 0, 0)),
            pl.BlockSpec((1, 1), lambda i: (0, 0)),
        ],
        compiler_params=pltpu.CompilerParams(
            vmem_limit_bytes=100 * 1024 * 1024),
        out_shape=[
            jax.ShapeDtypeStruct((_B * _N, _D_IN), jnp.float32),
            jax.ShapeDtypeStruct((_G, 1, _M), jnp.int32),
            jax.ShapeDtypeStruct((_G, 1, _M), jnp.int32),
            jax.ShapeDtypeStruct((_G, 1, _M), jnp.int32),
            jax.ShapeDtypeStruct((1, 1), jnp.float32),
        ],
    )(z2d, w_in, b_in2d, w_out, b_out2d, e0, e1, e2)
    return out, i0, i1, i2, loss


def kernel(z, W_in, b_in, W_out, b_out, emb0, emb1, emb2):
    z2d = z.reshape(_B * _N, _D_IN)
    out, i0, i1, i2, loss = _run(
        z2d, W_in, b_in.reshape(1, _PROJ), W_out, b_out.reshape(1, _D_IN),
        emb0, emb1, emb2)
    return (out.reshape(_B, _N, _D_IN),
            i0.reshape(_B, _N), i1.reshape(_B, _N), i2.reshape(_B, _N),
            loss[0, 0])
